# baseline (speedup 1.0000x reference)
import jax
import jax.numpy as jnp
from jax.experimental import pallas as pl
from jax.experimental.pallas import tpu as pltpu

NUM_EXPERTS = 16
TOP_K = 8
N_EMBD = 768
D_FFN = 384


def _moe_kernel(x_ref, rw_ref, w1_ref, w2_ref, out_ref, comb_ref, acc_ref,
                xbf_ref):
    e = pl.program_id(0)

    @pl.when(e == 0)
    def _routing():
        xt = x_ref[...]
        xbf_ref[...] = xt.astype(jnp.bfloat16)
        logits = jax.lax.dot_general(
            xt, rw_ref[...], (((1,), (1,)), ((), ())),
            preferred_element_type=jnp.float32)
        m = jnp.max(logits, axis=-1, keepdims=True)
        p = jnp.exp(logits - m)
        p = p / jnp.sum(p, axis=-1, keepdims=True)
        T = p.shape[0]
        col = jax.lax.broadcasted_iota(jnp.int32, (T, NUM_EXPERTS), 1)
        rank = jnp.zeros((T, NUM_EXPERTS), dtype=jnp.int32)
        for j in range(NUM_EXPERTS):
            pj = p[:, j:j + 1]
            beats = (pj > p) | ((pj == p) & (col > j))
            rank = rank + beats.astype(jnp.int32)
        w = jnp.where(rank < TOP_K, p, 0.0)
        w = w / jnp.sum(w, axis=-1, keepdims=True)
        comb_ref[...] = w
        acc_ref[...] = jnp.zeros_like(acc_ref)

    x = xbf_ref[...]
    h = jax.lax.dot_general(
        x, w1_ref[...].astype(jnp.bfloat16), (((1,), (0,)), ((), ())),
        preferred_element_type=jnp.float32)
    h = jax.nn.gelu(h).astype(jnp.bfloat16)
    y = jax.lax.dot_general(
        h, w2_ref[...].astype(jnp.bfloat16), (((1,), (0,)), ((), ())),
        preferred_element_type=jnp.float32)
    comb = comb_ref[...]
    col = jax.lax.broadcasted_iota(jnp.int32, comb.shape, 1)
    ce = jnp.sum(jnp.where(col == e, comb, 0.0), axis=1, keepdims=True)
    acc_ref[...] += ce * y

    @pl.when(e == NUM_EXPERTS - 1)
    def _finish():
        out_ref[...] = acc_ref[...]


def kernel(x, router_w, w1, w2):
    B, S, D = x.shape
    T = B * S
    xt = x.reshape(T, D)
    out = pl.pallas_call(
        _moe_kernel,
        grid=(NUM_EXPERTS,),
        in_specs=[
            pl.BlockSpec((T, D), lambda e: (0, 0)),
            pl.BlockSpec((NUM_EXPERTS, D), lambda e: (0, 0)),
            pl.BlockSpec((D, D_FFN), lambda e: (0, e)),
            pl.BlockSpec((D_FFN, D), lambda e: (e, 0)),
        ],
        out_specs=pl.BlockSpec((T, D), lambda e: (0, 0)),
        out_shape=jax.ShapeDtypeStruct((T, D), jnp.float32),
        scratch_shapes=[
            pltpu.VMEM((T, NUM_EXPERTS), jnp.float32),
            pltpu.VMEM((T, D), jnp.float32),
            pltpu.VMEM((T, D), jnp.bfloat16),
        ],
        compiler_params=pltpu.CompilerParams(
            dimension_semantics=("arbitrary",),
        ),
    )(xt, router_w, w1, w2)
    return out.reshape(B, S, D)


# speedup vs baseline: 1.3189x; 1.3189x over previous
import jax
import jax.numpy as jnp
from jax.experimental import pallas as pl
from jax.experimental.pallas import tpu as pltpu

NUM_EXPERTS = 16
TOP_K = 8
N_EMBD = 768
D_FFN = 384
BLK_T = 256


def _moe_kernel(x_ref, rw_ref, w1_ref, w2_ref, out_ref):
    xb = x_ref[...]
    logits = jax.lax.dot_general(
        xb, rw_ref[...], (((1,), (1,)), ((), ())),
        preferred_element_type=jnp.float32)
    col = jax.lax.broadcasted_iota(jnp.int32, logits.shape, 1)
    rank = jnp.zeros(logits.shape, dtype=jnp.int32)
    for j in range(NUM_EXPERTS):
        lj = logits[:, j:j + 1]
        beats = (lj > logits) | ((lj == logits) & (col > j))
        rank = rank + beats.astype(jnp.int32)
    sel = rank < TOP_K
    m = jnp.max(logits, axis=-1, keepdims=True)
    ew = jnp.where(sel, jnp.exp(logits - m), 0.0)
    comb = ew / jnp.sum(ew, axis=-1, keepdims=True)

    h = jax.lax.dot_general(
        xb, w1_ref[...], (((1,), (0,)), ((), ())),
        preferred_element_type=jnp.float32)
    h = jax.nn.gelu(h)
    g = jnp.concatenate(
        [h[:, e * D_FFN:(e + 1) * D_FFN] * comb[:, e:e + 1]
         for e in range(NUM_EXPERTS)], axis=1)
    out_ref[...] = jax.lax.dot_general(
        g, w2_ref[...], (((1,), (0,)), ((), ())),
        preferred_element_type=jnp.float32)


def kernel(x, router_w, w1, w2):
    B, S, D = x.shape
    T = B * S
    xt = x.reshape(T, D)
    EF = NUM_EXPERTS * D_FFN
    out = pl.pallas_call(
        _moe_kernel,
        grid=(T // BLK_T,),
        in_specs=[
            pl.BlockSpec((BLK_T, D), lambda t: (t, 0)),
            pl.BlockSpec((NUM_EXPERTS, D), lambda t: (0, 0)),
            pl.BlockSpec((D, EF), lambda t: (0, 0)),
            pl.BlockSpec((EF, D), lambda t: (0, 0)),
        ],
        out_specs=pl.BlockSpec((BLK_T, D), lambda t: (t, 0)),
        out_shape=jax.ShapeDtypeStruct((T, D), jnp.float32),
        compiler_params=pltpu.CompilerParams(
            dimension_semantics=("arbitrary",),
        ),
    )(xt, router_w, w1, w2)
    return out.reshape(B, S, D)
